# baseline (speedup 1.0000x reference)
import functools

import jax
import jax.numpy as jnp
from jax import lax
from jax.experimental import pallas as pl
from jax.experimental.pallas import tpu as pltpu
from jax.experimental.pallas import tpu_sc as plsc

N = 10000
F = 128
E = 320000
NC = 2
NS = 16
NW = NC * NS
CH = 128
EC = E // CH
ECP = 2504
FULLC = 80
TAILC = ECP - (NW - 1) * FULLC
NPAD = 10240
NPT = NPAD // NS
GRP = 8


_sc_mesh = plsc.VectorSubcoreMesh(core_axis_name="c", subcore_axis_name="s")
_sc_params = pltpu.CompilerParams(needs_layout_passes=False)


def _deg_body(dst2, zeros1d, deg_out, idx_v, hist_v):
    c = lax.axis_index("c")
    s = lax.axis_index("s")
    wid = s * NC + c
    base = wid * FULLC

    pltpu.sync_copy(zeros1d, hist_v)

    @pl.when(wid == NW - 1)
    def _():
        pltpu.sync_copy(dst2.at[pl.ds(base, TAILC)],
                        idx_v.at[pl.ds(0, TAILC)])

    @pl.when(wid != NW - 1)
    def _():
        pltpu.sync_copy(dst2.at[pl.ds(base, FULLC)], idx_v)

    nch = jnp.where(wid == NW - 1, TAILC, FULLC)
    ones16 = jnp.ones((16,), jnp.float32)

    def row_body(r, carry):
        for k in range(CH // 16):
            dv = idx_v[r, pl.ds(k * 16, 16)]
            plsc.addupdate_scatter(hist_v, [dv], ones16)
        return carry

    lax.fori_loop(0, nch, row_body, 0)

    pltpu.sync_copy(hist_v, deg_out.at[wid])


_deg_kernel = functools.partial(
    pl.kernel,
    out_type=jax.ShapeDtypeStruct((NW, NPAD), jnp.float32),
    mesh=_sc_mesh,
    compiler_params=_sc_params,
    scratch_types=[
        pltpu.VMEM((FULLC, CH), jnp.int32),
        pltpu.VMEM((NPAD,), jnp.float32),
    ],
)(_deg_body)


def _split_body(ei_ref, src_ref, dst_ref):
    i = pl.program_id(0)
    r = ei_ref[...]
    s_row = r[0:1, :].reshape(8, CH)
    d_row = r[1:2, :].reshape(8, CH)
    rows = i * 8 + lax.broadcasted_iota(jnp.int32, (8, CH), 0)
    dummy = N + lax.broadcasted_iota(jnp.int32, (8, CH), 1)
    ok = rows < EC
    src_ref[...] = jnp.where(ok, s_row, dummy)
    dst_ref[...] = jnp.where(ok, d_row, dummy)


def _split_call(ei):
    return pl.pallas_call(
        _split_body,
        grid=(ECP // 8,),
        in_specs=[pl.BlockSpec((2, 8 * CH), lambda i: (0, i))],
        out_specs=(
            pl.BlockSpec((8, CH), lambda i: (i, 0)),
            pl.BlockSpec((8, CH), lambda i: (i, 0)),
        ),
        out_shape=(
            jax.ShapeDtypeStruct((ECP, CH), jnp.int32),
            jax.ShapeDtypeStruct((ECP, CH), jnp.int32),
        ),
    )(ei)


_MBLK = 2048


def _mm_body(x_ref, w_ref, g_ref):
    g_ref[...] = lax.dot_general(x_ref[...], w_ref[...],
                                 (((1,), (1,)), ((), ())),
                                 preferred_element_type=jnp.float32)


def _mm_call(x, W):
    return pl.pallas_call(
        _mm_body,
        grid=(NPAD // _MBLK,),
        in_specs=[
            pl.BlockSpec((_MBLK, F), lambda i: (i, 0)),
            pl.BlockSpec((F, F), lambda i: (0, 0)),
        ],
        out_specs=pl.BlockSpec((_MBLK, F), lambda i: (i, 0)),
        out_shape=jax.ShapeDtypeStruct((NPAD, F), jnp.float32),
    )(x, W)


def _scale_body(deg_ref, g_ref, y_ref, dinv_ref):
    degsum = jnp.sum(deg_ref[...], axis=0, keepdims=True) + 1.0
    dinv_col = lax.rsqrt(degsum).reshape(_MBLK, 1)
    dinv_ref[...] = dinv_col
    y_ref[...] = g_ref[...] * dinv_col


def _scale_call(deg_p, g):
    return pl.pallas_call(
        _scale_body,
        grid=(NPAD // _MBLK,),
        in_specs=[
            pl.BlockSpec((NW, _MBLK), lambda i: (0, i)),
            pl.BlockSpec((_MBLK, F), lambda i: (i, 0)),
        ],
        out_specs=(
            pl.BlockSpec((_MBLK, F), lambda i: (i, 0)),
            pl.BlockSpec((_MBLK, 1), lambda i: (i, 0)),
        ),
        out_shape=(
            jax.ShapeDtypeStruct((NPAD, F), jnp.float32),
            jax.ShapeDtypeStruct((NPAD, 1), jnp.float32),
        ),
    )(deg_p, g)


def _prop_body(src2, dst2, y_hbm, zeros_blk, zp_out,
               src_v, dst_v, rows0, rows1, z_sp, gsem0, gsem1):
    c = lax.axis_index("c")
    s = lax.axis_index("s")
    wid = s * NC + c
    base = wid * FULLC

    pltpu.sync_copy(zeros_blk, rows0)
    for i in range(NPT // CH):
        pltpu.sync_copy(rows0, z_sp.at[pl.ds(s * NPT + i * CH, CH)])

    plsc.subcore_barrier()

    def run_group(gbase):
        pltpu.sync_copy(src2.at[pl.ds(gbase, GRP)], src_v)
        pltpu.sync_copy(dst2.at[pl.ds(gbase, GRP)], dst_v)
        pltpu.async_copy(y_hbm.at[src_v.at[0]], rows0, gsem0)

        def pair_body(t, carry2):
            j0 = t * 2
            j1 = j0 + 1
            pltpu.make_async_copy(y_hbm.at[src_v.at[j0]], rows0, gsem0).wait()
            pltpu.async_copy(y_hbm.at[src_v.at[j1]], rows1, gsem1)
            pltpu.sync_copy(rows0, z_sp.at[dst_v.at[j0]], add=True)
            pltpu.make_async_copy(y_hbm.at[src_v.at[j1]], rows1, gsem1).wait()

            @pl.when(j1 + 1 < GRP)
            def _():
                pltpu.async_copy(y_hbm.at[src_v.at[j1 + 1]], rows0, gsem0)

            pltpu.sync_copy(rows1, z_sp.at[dst_v.at[j1]], add=True)
            return carry2

        lax.fori_loop(0, GRP // 2, pair_body, 0)

    ngroups = jnp.where(wid == NW - 1, TAILC // GRP, FULLC // GRP)

    def group_body(g, carry):
        run_group(base + g * GRP)
        return carry

    lax.fori_loop(0, ngroups, group_body, 0)

    plsc.subcore_barrier()

    for i in range(NPT // CH):
        off = s * NPT + i * CH
        pltpu.sync_copy(z_sp.at[pl.ds(off, CH)], rows0)
        pltpu.sync_copy(rows0, zp_out.at[c, pl.ds(off, CH)])


_prop_kernel = functools.partial(
    pl.kernel,
    out_type=jax.ShapeDtypeStruct((NC, NPAD, F), jnp.float32),
    mesh=_sc_mesh,
    compiler_params=_sc_params,
    scratch_types=[
        pltpu.VMEM((GRP, CH), jnp.int32),
        pltpu.VMEM((GRP, CH), jnp.int32),
        pltpu.VMEM((CH, F), jnp.float32),
        pltpu.VMEM((CH, F), jnp.float32),
        pltpu.VMEM_SHARED((NPAD, F), jnp.float32),
        pltpu.SemaphoreType.DMA,
        pltpu.SemaphoreType.DMA,
    ],
)(_prop_body)


_DBLK = 2048


def _final_body(z_ref, y_ref, dinv_ref, x_ref, b_ref, o_ref):
    z = z_ref[0] + z_ref[1] + y_ref[...]
    o_ref[...] = x_ref[...] + jnp.maximum(
        z * dinv_ref[...] + b_ref[...], 0.0)


def _final_call(z_p, y_ext, dinv_col, x, b2):
    grid = NPAD // _DBLK
    return pl.pallas_call(
        _final_body,
        grid=(grid,),
        in_specs=[
            pl.BlockSpec((NC, _DBLK, F), lambda i: (0, i, 0)),
            pl.BlockSpec((_DBLK, F), lambda i: (i, 0)),
            pl.BlockSpec((_DBLK, 1), lambda i: (i, 0)),
            pl.BlockSpec((_DBLK, F), lambda i: (i, 0)),
            pl.BlockSpec((1, F), lambda i: (0, 0)),
        ],
        out_specs=pl.BlockSpec((_DBLK, F), lambda i: (i, 0)),
        out_shape=jax.ShapeDtypeStruct((N, F), jnp.float32),
    )(z_p, y_ext, dinv_col, x, b2)


def kernel(x, edge_index, W, b):
    zeros_blk = jnp.zeros((CH, F), jnp.float32)
    zeros1d = jnp.zeros((NPAD,), jnp.float32)
    b2 = b.reshape(1, F).astype(jnp.float32)
    x = x.astype(jnp.float32)

    src2, dst2 = _split_call(edge_index.astype(jnp.int32))
    g = _mm_call(x, W.astype(jnp.float32))
    deg_p = _deg_kernel(dst2, zeros1d)
    y_ext, dinv_col = _scale_call(deg_p, g)
    z_p = _prop_kernel(src2, dst2, y_ext, zeros_blk)
    return _final_call(z_p, y_ext, dinv_col, x, b2)


# speedup vs baseline: 1.6640x; 1.6640x over previous
import functools

import jax
import jax.numpy as jnp
from jax import lax
from jax.experimental import pallas as pl
from jax.experimental.pallas import tpu as pltpu
from jax.experimental.pallas import tpu_sc as plsc

N = 10000
F = 128
E = 320000
NC = 2
NS = 16
NW = NC * NS
CH = 128
EC = E // CH
ECP = 2560
FULLC = ECP // NW
NPAD = 10240
NPT = NPAD // NS
GRP = 8


_sc_mesh = plsc.VectorSubcoreMesh(core_axis_name="c", subcore_axis_name="s")
_sc_params = pltpu.CompilerParams(needs_layout_passes=False)


def _deg_body(dst2, zeros1d, deg_out, idx_v, hist_v):
    c = lax.axis_index("c")
    s = lax.axis_index("s")
    wid = s * NC + c
    base = wid * FULLC

    pltpu.sync_copy(zeros1d, hist_v)
    pltpu.sync_copy(dst2.at[pl.ds(base, FULLC)], idx_v)

    ones16 = jnp.ones((16,), jnp.float32)

    def row_body(r, carry):
        for k in range(CH // 16):
            dv = idx_v[r, pl.ds(k * 16, 16)]
            plsc.addupdate_scatter(hist_v, [dv], ones16)
        return carry

    lax.fori_loop(0, FULLC, row_body, 0)

    pltpu.sync_copy(hist_v, deg_out.at[wid])


_deg_kernel = functools.partial(
    pl.kernel,
    out_type=jax.ShapeDtypeStruct((NW, NPAD), jnp.float32),
    mesh=_sc_mesh,
    compiler_params=_sc_params,
    scratch_types=[
        pltpu.VMEM((FULLC, CH), jnp.int32),
        pltpu.VMEM((NPAD,), jnp.float32),
    ],
)(_deg_body)


_PBLK = 256


def _split_body(ei_ref, src_ref, dst_ref):
    i = pl.program_id(0)
    r = ei_ref[...]
    s_row = r[0:1, :].reshape(_PBLK, CH)
    d_row = r[1:2, :].reshape(_PBLK, CH)
    rows = i * _PBLK + lax.broadcasted_iota(jnp.int32, (_PBLK, CH), 0)
    dummy = N + lax.broadcasted_iota(jnp.int32, (_PBLK, CH), 1)
    ok = rows < EC
    src_ref[...] = jnp.where(ok, s_row, dummy)
    dst_ref[...] = jnp.where(ok, d_row, dummy)


def _split_call(ei):
    return pl.pallas_call(
        _split_body,
        grid=(ECP // _PBLK,),
        in_specs=[pl.BlockSpec((2, _PBLK * CH), lambda i: (0, i))],
        out_specs=(
            pl.BlockSpec((_PBLK, CH), lambda i: (i, 0)),
            pl.BlockSpec((_PBLK, CH), lambda i: (i, 0)),
        ),
        out_shape=(
            jax.ShapeDtypeStruct((ECP, CH), jnp.int32),
            jax.ShapeDtypeStruct((ECP, CH), jnp.int32),
        ),
    )(ei)


_MBLK = 2048


def _mm_body(x_ref, w_ref, g_ref):
    g_ref[...] = lax.dot_general(x_ref[...], w_ref[...],
                                 (((1,), (1,)), ((), ())),
                                 preferred_element_type=jnp.float32)


def _mm_call(x, W):
    return pl.pallas_call(
        _mm_body,
        grid=(NPAD // _MBLK,),
        in_specs=[
            pl.BlockSpec((_MBLK, F), lambda i: (i, 0)),
            pl.BlockSpec((F, F), lambda i: (0, 0)),
        ],
        out_specs=pl.BlockSpec((_MBLK, F), lambda i: (i, 0)),
        out_shape=jax.ShapeDtypeStruct((NPAD, F), jnp.float32),
    )(x, W)


def _scale_body(deg_ref, g_ref, y_ref, dinv_ref):
    degsum = jnp.sum(deg_ref[...], axis=0, keepdims=True) + 1.0
    dinv_col = lax.rsqrt(degsum).reshape(_MBLK, 1)
    dinv_ref[...] = dinv_col
    y_ref[...] = g_ref[...] * dinv_col


def _scale_call(deg_p, g):
    return pl.pallas_call(
        _scale_body,
        grid=(NPAD // _MBLK,),
        in_specs=[
            pl.BlockSpec((NW, _MBLK), lambda i: (0, i)),
            pl.BlockSpec((_MBLK, F), lambda i: (i, 0)),
        ],
        out_specs=(
            pl.BlockSpec((_MBLK, F), lambda i: (i, 0)),
            pl.BlockSpec((_MBLK, 1), lambda i: (i, 0)),
        ),
        out_shape=(
            jax.ShapeDtypeStruct((NPAD, F), jnp.float32),
            jax.ShapeDtypeStruct((NPAD, 1), jnp.float32),
        ),
    )(deg_p, g)


def _prop_body(src2, dst2, y_hbm, zeros_blk, zp_out,
               src_v, dst_v, rows0, rows1, z_sp, gsem0, gsem1):
    c = lax.axis_index("c")
    s = lax.axis_index("s")
    wid = s * NC + c
    base = wid * FULLC

    pltpu.sync_copy(zeros_blk, rows0)
    for i in range(NPT // CH):
        pltpu.sync_copy(rows0, z_sp.at[pl.ds(s * NPT + i * CH, CH)])

    plsc.subcore_barrier()

    def run_group(gbase):
        pltpu.sync_copy(src2.at[pl.ds(gbase, GRP)], src_v)
        pltpu.sync_copy(dst2.at[pl.ds(gbase, GRP)], dst_v)
        pltpu.async_copy(y_hbm.at[src_v.at[0]], rows0, gsem0)

        def pair_body(t, carry2):
            j0 = t * 2
            j1 = j0 + 1
            pltpu.make_async_copy(y_hbm.at[src_v.at[j0]], rows0, gsem0).wait()
            pltpu.async_copy(y_hbm.at[src_v.at[j1]], rows1, gsem1)
            pltpu.sync_copy(rows0, z_sp.at[dst_v.at[j0]], add=True)
            pltpu.make_async_copy(y_hbm.at[src_v.at[j1]], rows1, gsem1).wait()

            @pl.when(j1 + 1 < GRP)
            def _():
                pltpu.async_copy(y_hbm.at[src_v.at[j1 + 1]], rows0, gsem0)

            pltpu.sync_copy(rows1, z_sp.at[dst_v.at[j1]], add=True)
            return carry2

        lax.fori_loop(0, GRP // 2, pair_body, 0)

    def group_body(g, carry):
        run_group(base + g * GRP)
        return carry

    lax.fori_loop(0, FULLC // GRP, group_body, 0)

    plsc.subcore_barrier()

    for i in range(NPT // CH):
        off = s * NPT + i * CH
        pltpu.sync_copy(z_sp.at[pl.ds(off, CH)], rows0)
        pltpu.sync_copy(rows0, zp_out.at[c, pl.ds(off, CH)])


_prop_kernel = functools.partial(
    pl.kernel,
    out_type=jax.ShapeDtypeStruct((NC, NPAD, F), jnp.float32),
    mesh=_sc_mesh,
    compiler_params=_sc_params,
    scratch_types=[
        pltpu.VMEM((GRP, CH), jnp.int32),
        pltpu.VMEM((GRP, CH), jnp.int32),
        pltpu.VMEM((CH, F), jnp.float32),
        pltpu.VMEM((CH, F), jnp.float32),
        pltpu.VMEM_SHARED((NPAD, F), jnp.float32),
        pltpu.SemaphoreType.DMA,
        pltpu.SemaphoreType.DMA,
    ],
)(_prop_body)


_DBLK = 2048


def _final_body(z_ref, y_ref, dinv_ref, x_ref, b_ref, o_ref):
    z = z_ref[0] + z_ref[1] + y_ref[...]
    o_ref[...] = x_ref[...] + jnp.maximum(
        z * dinv_ref[...] + b_ref[...], 0.0)


def _final_call(z_p, y_ext, dinv_col, x, b2):
    grid = NPAD // _DBLK
    return pl.pallas_call(
        _final_body,
        grid=(grid,),
        in_specs=[
            pl.BlockSpec((NC, _DBLK, F), lambda i: (0, i, 0)),
            pl.BlockSpec((_DBLK, F), lambda i: (i, 0)),
            pl.BlockSpec((_DBLK, 1), lambda i: (i, 0)),
            pl.BlockSpec((_DBLK, F), lambda i: (i, 0)),
            pl.BlockSpec((1, F), lambda i: (0, 0)),
        ],
        out_specs=pl.BlockSpec((_DBLK, F), lambda i: (i, 0)),
        out_shape=jax.ShapeDtypeStruct((N, F), jnp.float32),
    )(z_p, y_ext, dinv_col, x, b2)


def kernel(x, edge_index, W, b):
    zeros_blk = jnp.zeros((CH, F), jnp.float32)
    zeros1d = jnp.zeros((NPAD,), jnp.float32)
    b2 = b.reshape(1, F).astype(jnp.float32)
    x = x.astype(jnp.float32)

    src2, dst2 = _split_call(edge_index.astype(jnp.int32))
    g = _mm_call(x, W.astype(jnp.float32))
    deg_p = _deg_kernel(dst2, zeros1d)
    y_ext, dinv_col = _scale_call(deg_p, g)
    z_p = _prop_kernel(src2, dst2, y_ext, zeros_blk)
    return _final_call(z_p, y_ext, dinv_col, x, b2)


# speedup vs baseline: 1.7326x; 1.0412x over previous
import functools

import jax
import jax.numpy as jnp
from jax import lax
from jax.experimental import pallas as pl
from jax.experimental.pallas import tpu as pltpu
from jax.experimental.pallas import tpu_sc as plsc

N = 10000
F = 128
E = 320000
NC = 2
NS = 16
NW = NC * NS
CH = 128
EC = E // CH
ECP = 2560
FULLC = ECP // NW
NPAD = 10240
NPT = NPAD // NS
GRP = 16


_sc_mesh = plsc.VectorSubcoreMesh(core_axis_name="c", subcore_axis_name="s")
_sc_params = pltpu.CompilerParams(needs_layout_passes=False)


def _deg_body(dst2, zeros1d, deg_out, idx_v, hist_v):
    c = lax.axis_index("c")
    s = lax.axis_index("s")
    wid = s * NC + c
    base = wid * FULLC

    pltpu.sync_copy(zeros1d, hist_v)
    pltpu.sync_copy(dst2.at[pl.ds(base, FULLC)], idx_v)

    ones16 = jnp.ones((16,), jnp.float32)

    def row_body(r, carry):
        for k in range(CH // 16):
            dv = idx_v[r, pl.ds(k * 16, 16)]
            plsc.addupdate_scatter(hist_v, [dv], ones16)
        return carry

    lax.fori_loop(0, FULLC, row_body, 0)

    pltpu.sync_copy(hist_v, deg_out.at[wid])


_deg_kernel = functools.partial(
    pl.kernel,
    out_type=jax.ShapeDtypeStruct((NW, NPAD), jnp.float32),
    mesh=_sc_mesh,
    compiler_params=_sc_params,
    scratch_types=[
        pltpu.VMEM((FULLC, CH), jnp.int32),
        pltpu.VMEM((NPAD,), jnp.float32),
    ],
)(_deg_body)


_PBLK = 256


def _split_body(ei_ref, src_ref, dst_ref):
    i = pl.program_id(0)
    r = ei_ref[...]
    s_row = r[0:1, :].reshape(_PBLK, CH)
    d_row = r[1:2, :].reshape(_PBLK, CH)
    rows = i * _PBLK + lax.broadcasted_iota(jnp.int32, (_PBLK, CH), 0)
    dummy = N + lax.broadcasted_iota(jnp.int32, (_PBLK, CH), 1)
    ok = rows < EC
    src_ref[...] = jnp.where(ok, s_row, dummy)
    dst_ref[...] = jnp.where(ok, d_row, dummy)


def _split_call(ei):
    return pl.pallas_call(
        _split_body,
        grid=(ECP // _PBLK,),
        in_specs=[pl.BlockSpec((2, _PBLK * CH), lambda i: (0, i))],
        out_specs=(
            pl.BlockSpec((_PBLK, CH), lambda i: (i, 0)),
            pl.BlockSpec((_PBLK, CH), lambda i: (i, 0)),
        ),
        out_shape=(
            jax.ShapeDtypeStruct((ECP, CH), jnp.int32),
            jax.ShapeDtypeStruct((ECP, CH), jnp.int32),
        ),
    )(ei)


_MBLK = 2048


def _mm_body(x_ref, w_ref, g_ref):
    g_ref[...] = lax.dot_general(x_ref[...], w_ref[...],
                                 (((1,), (1,)), ((), ())),
                                 preferred_element_type=jnp.float32)


def _mm_call(x, W):
    return pl.pallas_call(
        _mm_body,
        grid=(NPAD // _MBLK,),
        in_specs=[
            pl.BlockSpec((_MBLK, F), lambda i: (i, 0)),
            pl.BlockSpec((F, F), lambda i: (0, 0)),
        ],
        out_specs=pl.BlockSpec((_MBLK, F), lambda i: (i, 0)),
        out_shape=jax.ShapeDtypeStruct((NPAD, F), jnp.float32),
    )(x, W)


def _scale_body(deg_ref, g_ref, y_ref, dinv_ref):
    degsum = jnp.sum(deg_ref[...], axis=0, keepdims=True) + 1.0
    dinv_col = lax.rsqrt(degsum).reshape(_MBLK, 1)
    dinv_ref[...] = dinv_col
    y_ref[...] = g_ref[...] * dinv_col


def _scale_call(deg_p, g):
    return pl.pallas_call(
        _scale_body,
        grid=(NPAD // _MBLK,),
        in_specs=[
            pl.BlockSpec((NW, _MBLK), lambda i: (0, i)),
            pl.BlockSpec((_MBLK, F), lambda i: (i, 0)),
        ],
        out_specs=(
            pl.BlockSpec((_MBLK, F), lambda i: (i, 0)),
            pl.BlockSpec((_MBLK, 1), lambda i: (i, 0)),
        ),
        out_shape=(
            jax.ShapeDtypeStruct((NPAD, F), jnp.float32),
            jax.ShapeDtypeStruct((NPAD, 1), jnp.float32),
        ),
    )(deg_p, g)


def _prop_body(src2, dst2, y_hbm, zeros_blk, zp_out,
               src_v, dst_v, rows0, rows1, z_sp, gsem0, gsem1):
    c = lax.axis_index("c")
    s = lax.axis_index("s")
    wid = s * NC + c
    base = wid * FULLC

    pltpu.sync_copy(zeros_blk, rows0)
    for i in range(NPT // CH):
        pltpu.sync_copy(rows0, z_sp.at[pl.ds(s * NPT + i * CH, CH)])

    plsc.subcore_barrier()

    def run_group(gbase):
        pltpu.sync_copy(src2.at[pl.ds(gbase, GRP)], src_v)
        pltpu.sync_copy(dst2.at[pl.ds(gbase, GRP)], dst_v)
        pltpu.async_copy(y_hbm.at[src_v.at[0]], rows0, gsem0)

        def pair_body(t, carry2):
            j0 = t * 2
            j1 = j0 + 1
            pltpu.make_async_copy(y_hbm.at[src_v.at[j0]], rows0, gsem0).wait()
            pltpu.async_copy(y_hbm.at[src_v.at[j1]], rows1, gsem1)
            pltpu.sync_copy(rows0, z_sp.at[dst_v.at[j0]], add=True)
            pltpu.make_async_copy(y_hbm.at[src_v.at[j1]], rows1, gsem1).wait()

            @pl.when(j1 + 1 < GRP)
            def _():
                pltpu.async_copy(y_hbm.at[src_v.at[j1 + 1]], rows0, gsem0)

            pltpu.sync_copy(rows1, z_sp.at[dst_v.at[j1]], add=True)
            return carry2

        lax.fori_loop(0, GRP // 2, pair_body, 0)

    def group_body(g, carry):
        run_group(base + g * GRP)
        return carry

    lax.fori_loop(0, FULLC // GRP, group_body, 0)

    plsc.subcore_barrier()

    for i in range(NPT // CH):
        off = s * NPT + i * CH
        pltpu.sync_copy(z_sp.at[pl.ds(off, CH)], rows0)
        pltpu.sync_copy(rows0, zp_out.at[c, pl.ds(off, CH)])


_prop_kernel = functools.partial(
    pl.kernel,
    out_type=jax.ShapeDtypeStruct((NC, NPAD, F), jnp.float32),
    mesh=_sc_mesh,
    compiler_params=_sc_params,
    scratch_types=[
        pltpu.VMEM((GRP, CH), jnp.int32),
        pltpu.VMEM((GRP, CH), jnp.int32),
        pltpu.VMEM((CH, F), jnp.float32),
        pltpu.VMEM((CH, F), jnp.float32),
        pltpu.VMEM_SHARED((NPAD, F), jnp.float32),
        pltpu.SemaphoreType.DMA,
        pltpu.SemaphoreType.DMA,
    ],
)(_prop_body)


_DBLK = 2048


def _final_body(z_ref, y_ref, dinv_ref, x_ref, b_ref, o_ref):
    z = z_ref[0] + z_ref[1] + y_ref[...]
    o_ref[...] = x_ref[...] + jnp.maximum(
        z * dinv_ref[...] + b_ref[...], 0.0)


def _final_call(z_p, y_ext, dinv_col, x, b2):
    grid = NPAD // _DBLK
    return pl.pallas_call(
        _final_body,
        grid=(grid,),
        in_specs=[
            pl.BlockSpec((NC, _DBLK, F), lambda i: (0, i, 0)),
            pl.BlockSpec((_DBLK, F), lambda i: (i, 0)),
            pl.BlockSpec((_DBLK, 1), lambda i: (i, 0)),
            pl.BlockSpec((_DBLK, F), lambda i: (i, 0)),
            pl.BlockSpec((1, F), lambda i: (0, 0)),
        ],
        out_specs=pl.BlockSpec((_DBLK, F), lambda i: (i, 0)),
        out_shape=jax.ShapeDtypeStruct((N, F), jnp.float32),
    )(z_p, y_ext, dinv_col, x, b2)


def kernel(x, edge_index, W, b):
    zeros_blk = jnp.zeros((CH, F), jnp.float32)
    zeros1d = jnp.zeros((NPAD,), jnp.float32)
    b2 = b.reshape(1, F).astype(jnp.float32)
    x = x.astype(jnp.float32)

    src2, dst2 = _split_call(edge_index.astype(jnp.int32))
    g = _mm_call(x, W.astype(jnp.float32))
    deg_p = _deg_kernel(dst2, zeros1d)
    y_ext, dinv_col = _scale_call(deg_p, g)
    z_p = _prop_kernel(src2, dst2, y_ext, zeros_blk)
    return _final_call(z_p, y_ext, dinv_col, x, b2)


# speedup vs baseline: 1.7811x; 1.0279x over previous
import functools

import jax
import jax.numpy as jnp
from jax import lax
from jax.experimental import pallas as pl
from jax.experimental.pallas import tpu as pltpu
from jax.experimental.pallas import tpu_sc as plsc

N = 10000
F = 128
E = 320000
NC = 2
NS = 16
NW = NC * NS
CH = 128
EC = E // CH
ECP = 2560
FULLC = ECP // NW
NPAD = 10240
NPT = NPAD // NS
GRP = 40


_sc_mesh = plsc.VectorSubcoreMesh(core_axis_name="c", subcore_axis_name="s")
_sc_params = pltpu.CompilerParams(needs_layout_passes=False)


def _deg_body(dst2, zeros1d, deg_out, idx_v, hist_v):
    c = lax.axis_index("c")
    s = lax.axis_index("s")
    wid = s * NC + c
    base = wid * FULLC

    pltpu.sync_copy(zeros1d, hist_v)
    pltpu.sync_copy(dst2.at[pl.ds(base, FULLC)], idx_v)

    ones16 = jnp.ones((16,), jnp.float32)

    def row_body(r, carry):
        for k in range(CH // 16):
            dv = idx_v[r, pl.ds(k * 16, 16)]
            plsc.addupdate_scatter(hist_v, [dv], ones16)
        return carry

    lax.fori_loop(0, FULLC, row_body, 0)

    pltpu.sync_copy(hist_v, deg_out.at[wid])


_deg_kernel = functools.partial(
    pl.kernel,
    out_type=jax.ShapeDtypeStruct((NW, NPAD), jnp.float32),
    mesh=_sc_mesh,
    compiler_params=_sc_params,
    scratch_types=[
        pltpu.VMEM((FULLC, CH), jnp.int32),
        pltpu.VMEM((NPAD,), jnp.float32),
    ],
)(_deg_body)


_PBLK = 256


def _split_body(ei_ref, src_ref, dst_ref):
    i = pl.program_id(0)
    r = ei_ref[...]
    s_row = r[0:1, :].reshape(_PBLK, CH)
    d_row = r[1:2, :].reshape(_PBLK, CH)
    rows = i * _PBLK + lax.broadcasted_iota(jnp.int32, (_PBLK, CH), 0)
    dummy = N + lax.broadcasted_iota(jnp.int32, (_PBLK, CH), 1)
    ok = rows < EC
    src_ref[...] = jnp.where(ok, s_row, dummy)
    dst_ref[...] = jnp.where(ok, d_row, dummy)


def _split_call(ei):
    return pl.pallas_call(
        _split_body,
        grid=(ECP // _PBLK,),
        in_specs=[pl.BlockSpec((2, _PBLK * CH), lambda i: (0, i))],
        out_specs=(
            pl.BlockSpec((_PBLK, CH), lambda i: (i, 0)),
            pl.BlockSpec((_PBLK, CH), lambda i: (i, 0)),
        ),
        out_shape=(
            jax.ShapeDtypeStruct((ECP, CH), jnp.int32),
            jax.ShapeDtypeStruct((ECP, CH), jnp.int32),
        ),
    )(ei)


_MBLK = 2048


def _mm_body(x_ref, w_ref, g_ref):
    g_ref[...] = lax.dot_general(x_ref[...], w_ref[...],
                                 (((1,), (1,)), ((), ())),
                                 preferred_element_type=jnp.float32)


def _mm_call(x, W):
    return pl.pallas_call(
        _mm_body,
        grid=(NPAD // _MBLK,),
        in_specs=[
            pl.BlockSpec((_MBLK, F), lambda i: (i, 0)),
            pl.BlockSpec((F, F), lambda i: (0, 0)),
        ],
        out_specs=pl.BlockSpec((_MBLK, F), lambda i: (i, 0)),
        out_shape=jax.ShapeDtypeStruct((NPAD, F), jnp.float32),
    )(x, W)


def _scale_body(deg_ref, g_ref, y_ref, dinv_ref):
    degsum = jnp.sum(deg_ref[...], axis=0, keepdims=True) + 1.0
    dinv_col = lax.rsqrt(degsum).reshape(_MBLK, 1)
    dinv_ref[...] = dinv_col
    y_ref[...] = g_ref[...] * dinv_col


def _scale_call(deg_p, g):
    return pl.pallas_call(
        _scale_body,
        grid=(NPAD // _MBLK,),
        in_specs=[
            pl.BlockSpec((NW, _MBLK), lambda i: (0, i)),
            pl.BlockSpec((_MBLK, F), lambda i: (i, 0)),
        ],
        out_specs=(
            pl.BlockSpec((_MBLK, F), lambda i: (i, 0)),
            pl.BlockSpec((_MBLK, 1), lambda i: (i, 0)),
        ),
        out_shape=(
            jax.ShapeDtypeStruct((NPAD, F), jnp.float32),
            jax.ShapeDtypeStruct((NPAD, 1), jnp.float32),
        ),
    )(deg_p, g)


def _prop_body(src2, dst2, y_hbm, zeros_blk, zp_out,
               src_v, dst_v, rows0, rows1, z_sp, gsem0, gsem1):
    c = lax.axis_index("c")
    s = lax.axis_index("s")
    wid = s * NC + c
    base = wid * FULLC

    pltpu.sync_copy(zeros_blk, rows0)
    for i in range(NPT // CH):
        pltpu.sync_copy(rows0, z_sp.at[pl.ds(s * NPT + i * CH, CH)])

    plsc.subcore_barrier()

    def run_group(gbase):
        pltpu.sync_copy(src2.at[pl.ds(gbase, GRP)], src_v)
        pltpu.sync_copy(dst2.at[pl.ds(gbase, GRP)], dst_v)
        pltpu.async_copy(y_hbm.at[src_v.at[0]], rows0, gsem0)

        def pair_body(t, carry2):
            j0 = t * 2
            j1 = j0 + 1
            pltpu.make_async_copy(y_hbm.at[src_v.at[j0]], rows0, gsem0).wait()
            pltpu.async_copy(y_hbm.at[src_v.at[j1]], rows1, gsem1)
            pltpu.sync_copy(rows0, z_sp.at[dst_v.at[j0]], add=True)
            pltpu.make_async_copy(y_hbm.at[src_v.at[j1]], rows1, gsem1).wait()

            @pl.when(j1 + 1 < GRP)
            def _():
                pltpu.async_copy(y_hbm.at[src_v.at[j1 + 1]], rows0, gsem0)

            pltpu.sync_copy(rows1, z_sp.at[dst_v.at[j1]], add=True)
            return carry2

        lax.fori_loop(0, GRP // 2, pair_body, 0)

    def group_body(g, carry):
        run_group(base + g * GRP)
        return carry

    lax.fori_loop(0, FULLC // GRP, group_body, 0)

    plsc.subcore_barrier()

    for i in range(NPT // CH):
        off = s * NPT + i * CH
        pltpu.sync_copy(z_sp.at[pl.ds(off, CH)], rows0)
        pltpu.sync_copy(rows0, zp_out.at[c, pl.ds(off, CH)])


_prop_kernel = functools.partial(
    pl.kernel,
    out_type=jax.ShapeDtypeStruct((NC, NPAD, F), jnp.float32),
    mesh=_sc_mesh,
    compiler_params=_sc_params,
    scratch_types=[
        pltpu.VMEM((GRP, CH), jnp.int32),
        pltpu.VMEM((GRP, CH), jnp.int32),
        pltpu.VMEM((CH, F), jnp.float32),
        pltpu.VMEM((CH, F), jnp.float32),
        pltpu.VMEM_SHARED((NPAD, F), jnp.float32),
        pltpu.SemaphoreType.DMA,
        pltpu.SemaphoreType.DMA,
    ],
)(_prop_body)


_DBLK = 2048


def _final_body(z_ref, y_ref, dinv_ref, x_ref, b_ref, o_ref):
    z = z_ref[0] + z_ref[1] + y_ref[...]
    o_ref[...] = x_ref[...] + jnp.maximum(
        z * dinv_ref[...] + b_ref[...], 0.0)


def _final_call(z_p, y_ext, dinv_col, x, b2):
    grid = NPAD // _DBLK
    return pl.pallas_call(
        _final_body,
        grid=(grid,),
        in_specs=[
            pl.BlockSpec((NC, _DBLK, F), lambda i: (0, i, 0)),
            pl.BlockSpec((_DBLK, F), lambda i: (i, 0)),
            pl.BlockSpec((_DBLK, 1), lambda i: (i, 0)),
            pl.BlockSpec((_DBLK, F), lambda i: (i, 0)),
            pl.BlockSpec((1, F), lambda i: (0, 0)),
        ],
        out_specs=pl.BlockSpec((_DBLK, F), lambda i: (i, 0)),
        out_shape=jax.ShapeDtypeStruct((N, F), jnp.float32),
    )(z_p, y_ext, dinv_col, x, b2)


def kernel(x, edge_index, W, b):
    zeros_blk = jnp.zeros((CH, F), jnp.float32)
    zeros1d = jnp.zeros((NPAD,), jnp.float32)
    b2 = b.reshape(1, F).astype(jnp.float32)
    x = x.astype(jnp.float32)

    src2, dst2 = _split_call(edge_index.astype(jnp.int32))
    g = _mm_call(x, W.astype(jnp.float32))
    deg_p = _deg_kernel(dst2, zeros1d)
    y_ext, dinv_col = _scale_call(deg_p, g)
    z_p = _prop_kernel(src2, dst2, y_ext, zeros_blk)
    return _final_call(z_p, y_ext, dinv_col, x, b2)
